# Initial kernel scaffold; baseline (speedup 1.0000x reference)
import functools

import jax
import jax.numpy as jnp
from jax import lax
from jax.experimental import pallas as pl
from jax.experimental.pallas import tpu as pltpu
from jax.experimental.pallas import tpu_sc as plsc

f32 = jnp.float32
i32 = jnp.int32

N = 10000
E = 320000
NR = 30080
NR2 = 10112
LEN1 = 655360
LEN2 = 327680
GROUP = 40
DEPTH = 4
STRIDE1 = NR // 16
STRIDE2 = NR2 // 16

_mesh = plsc.VectorSubcoreMesh(
    core_axis_name="c", subcore_axis_name="s", num_cores=2, num_subcores=16)


def _mm_body(xb, wb, ob):
    ob[...] = jnp.dot(xb[...], wb[...], preferred_element_type=f32)


def _tc_matmul(x, wcat):
    return pl.pallas_call(
        _mm_body,
        grid=(10,),
        in_specs=[pl.BlockSpec((1000, 128), lambda i: (i, 0)),
                  pl.BlockSpec((128, 384), lambda i: (0, 0))],
        out_specs=pl.BlockSpec((1000, 384), lambda i: (i, 0)),
        out_shape=jax.ShapeDtypeStruct((N, 384), f32),
    )(x, wcat)


def _idx_body(sb, db, rb, asb, adb, bsb, bdb):
    sv, dv, rv = sb[...], db[...], rb[...]
    asb[...] = sv * 3 + 2 * rv
    adb[...] = dv * 3 + 2 * rv
    bsb[...] = sv * 3 + 1
    bdb[...] = dv * 3 + 1


def _tc_indices(srcm, dstm, revm):
    spec = pl.BlockSpec((250, 128), lambda i: (i, 0))
    sh = jax.ShapeDtypeStruct((2500, 128), i32)
    return pl.pallas_call(
        _idx_body,
        grid=(10,),
        in_specs=[spec, spec, spec],
        out_specs=[spec, spec, spec, spec],
        out_shape=[sh, sh, sh, sh],
    )(srcm, dstm, revm)


def _scale_body(cb, hb, lob, hib, db):
    cnt = cb[0] + cb[1]
    dis = lax.rsqrt(cnt + 1.0)
    db[...] = dis
    g = hb[...] * dis
    lob[...] = g[:, :64]
    hib[...] = g[:, 64:]


def _tc_scale(cnt, h3p):
    return pl.pallas_call(
        _scale_body,
        grid=(NR // 128,),
        in_specs=[pl.BlockSpec((2, 128, 1), lambda i: (0, i, 0)),
                  pl.BlockSpec((128, 128), lambda i: (i, 0))],
        out_specs=[pl.BlockSpec((128, 64), lambda i: (i, 0)),
                   pl.BlockSpec((128, 64), lambda i: (i, 0)),
                   pl.BlockSpec((128, 1), lambda i: (i, 0))],
        out_shape=[jax.ShapeDtypeStruct((NR, 64), f32),
                   jax.ShapeDtypeStruct((NR, 64), f32),
                   jax.ShapeDtypeStruct((NR, 1), f32)],
    )(cnt, h3p)


def _l2_body(lob, hib, hb, db, bcb, wb, pb, qb):
    acc = jnp.zeros((128, 48), f32)
    for c in range(3):
        a = jnp.concatenate([lob[:, c, :], hib[:, c, :]], axis=1)
        dd = db[:, c, :]
        hc = jnp.maximum(dd * a + dd * dd * hb[:, c, :] + bcb[c][None, :], 0.0)
        acc = acc + jnp.dot(hc, wb[c], preferred_element_type=f32)
    pb[...] = acc
    qb[...] = db[:, 1, :] * acc


def _tc_l2(agglo3, agghi3, h33, dis3, bcat, w2p):
    sh48 = jax.ShapeDtypeStruct((NR2, 48), f32)
    return pl.pallas_call(
        _l2_body,
        grid=(NR2 // 128,),
        in_specs=[pl.BlockSpec((128, 3, 64), lambda i: (i, 0, 0)),
                  pl.BlockSpec((128, 3, 64), lambda i: (i, 0, 0)),
                  pl.BlockSpec((128, 3, 128), lambda i: (i, 0, 0)),
                  pl.BlockSpec((128, 3, 1), lambda i: (i, 0, 0)),
                  pl.BlockSpec((3, 128), lambda i: (0, 0)),
                  pl.BlockSpec((3, 128, 48), lambda i: (0, 0, 0))],
        out_specs=[pl.BlockSpec((128, 48), lambda i: (i, 0)),
                   pl.BlockSpec((128, 48), lambda i: (i, 0))],
        out_shape=[sh48, sh48],
    )(agglo3, agghi3, h33, dis3, bcat, w2p)


def _out_body(ptb, pb, db, b2b, ob):
    s = ptb[0] + ptb[1]
    dd = db[...]
    o = dd * s[:, :40] + dd * dd * pb[:, :40] + b2b[...]
    m = jnp.max(o, axis=1, keepdims=True)
    z = jnp.sum(jnp.exp(o - m), axis=1, keepdims=True)
    ob[...] = o - m - jnp.log(z)


def _tc_out(parts, p, disall, b2p):
    return pl.pallas_call(
        _out_body,
        grid=(NR2 // 128,),
        in_specs=[pl.BlockSpec((2, 128, 48), lambda i: (0, i, 0)),
                  pl.BlockSpec((128, 48), lambda i: (i, 0)),
                  pl.BlockSpec((128, 1), lambda i: (i, 0)),
                  pl.BlockSpec((1, 40), lambda i: (0, 0))],
        out_specs=pl.BlockSpec((128, 40), lambda i: (i, 0)),
        out_shape=jax.ShapeDtypeStruct((NR2, 40), f32),
    )(parts, p, disall, b2p)


def _hist_body(dst3, ones_h, z_h, out, idxd, ones_v, stage, hist, sem):
    cid = lax.axis_index("c")
    sid = lax.axis_index("s")
    pltpu.sync_copy(z_h, stage)
    pltpu.sync_copy(stage, hist.at[pl.ds(sid * STRIDE1, STRIDE1)])
    pltpu.sync_copy(ones_h, ones_v)
    plsc.subcore_barrier()
    base = (cid * 16 + sid) * 160
    for g in range(4):
        pltpu.sync_copy(dst3.at[pl.ds(base + g * GROUP, GROUP)], idxd)

        def it_body(it, carry):
            for b in range(8):
                pltpu.async_copy(ones_v, hist.at[idxd.at[it * 8 + b]], sem,
                                 add=True)
            for b in range(8):
                pltpu.make_async_copy(ones_v, hist.at[idxd.at[0]], sem).wait()
            return carry

        lax.fori_loop(0, 5, it_body, 0)
    plsc.subcore_barrier()
    pltpu.sync_copy(hist.at[pl.ds(sid * STRIDE1, STRIDE1)], stage)
    pltpu.sync_copy(stage, out.at[cid, sid])


def _sc_hist(dst3, ones_h, z_h):
    return pl.kernel(
        _hist_body,
        out_type=jax.ShapeDtypeStruct((2, 16, STRIDE1), f32),
        mesh=_mesh,
        scratch_types=[
            pltpu.VMEM((GROUP, 128), i32),
            pltpu.VMEM((128,), f32),
            pltpu.VMEM((STRIDE1,), f32),
            pltpu.VMEM_SHARED((NR,), f32),
            pltpu.SemaphoreType.DMA,
        ],
    )(dst3, ones_h, z_h)


def _row_pipeline(src_hbm, dst_hbm, table, acc, idxg, idxd, rows, sem_g,
                  sem_s, base, groups):
    for g in range(groups):
        if g > 0:
            for b in range(DEPTH):
                pltpu.make_async_copy(
                    rows.at[b], acc.at[idxd.at[0]], sem_s.at[b]).wait()
        pltpu.sync_copy(src_hbm.at[pl.ds(base + g * GROUP, GROUP)], idxg)
        pltpu.sync_copy(dst_hbm.at[pl.ds(base + g * GROUP, GROUP)], idxd)

        def it_body(it, carry):
            for b in range(DEPTH):
                @pl.when(it > 0)
                def _drain(b=b):
                    pltpu.make_async_copy(
                        rows.at[b], acc.at[idxd.at[0]], sem_s.at[b]).wait()
                pltpu.async_copy(
                    table.at[idxg.at[it * DEPTH + b]], rows.at[b],
                    sem_g.at[b])
            for b in range(DEPTH):
                pltpu.make_async_copy(
                    table.at[idxg.at[0]], rows.at[b], sem_g.at[b]).wait()
                pltpu.async_copy(
                    rows.at[b], acc.at[idxd.at[it * DEPTH + b]],
                    sem_s.at[b], add=True)
            return carry

        lax.fori_loop(0, GROUP // DEPTH, it_body, 0)
    for b in range(DEPTH):
        pltpu.make_async_copy(rows.at[b], acc.at[idxd.at[0]], sem_s.at[b]).wait()


def _agg1_body(src3, dst3, glo, ghi, z64, out, idxg, idxd, rows, st128, st88,
               acc, sem_g, sem_s):
    cid = lax.axis_index("c")
    sid = lax.axis_index("s")
    pltpu.sync_copy(z64, st128)
    pltpu.sync_copy(z64.at[pl.ds(0, 88)], st88)
    r0 = sid * STRIDE1
    for k in range(14):
        pltpu.sync_copy(st128, acc.at[pl.ds(r0 + k * 128, 128)])
    pltpu.sync_copy(st88, acc.at[pl.ds(r0 + 1792, 88)])
    plsc.subcore_barrier()
    base = sid * 320
    pl.when(cid == 0)(lambda: _row_pipeline(
        src3, dst3, glo, acc, idxg, idxd, rows, sem_g, sem_s, base, 8))
    pl.when(cid == 1)(lambda: _row_pipeline(
        src3, dst3, ghi, acc, idxg, idxd, rows, sem_g, sem_s, base, 8))
    plsc.subcore_barrier()
    for k in range(14):
        pltpu.sync_copy(acc.at[pl.ds(r0 + k * 128, 128)], st128)
        pltpu.sync_copy(st128, out.at[cid, pl.ds(r0 + k * 128, 128)])
    pltpu.sync_copy(acc.at[pl.ds(r0 + 1792, 88)], st88)
    pltpu.sync_copy(st88, out.at[cid, pl.ds(r0 + 1792, 88)])


def _sc_agg1(src3, dst3, glo, ghi, z64):
    return pl.kernel(
        _agg1_body,
        out_type=jax.ShapeDtypeStruct((2, NR, 64), f32),
        mesh=_mesh,
        scratch_types=[
            pltpu.VMEM((GROUP, 128), i32),
            pltpu.VMEM((GROUP, 128), i32),
            pltpu.VMEM((DEPTH, 128, 64), f32),
            pltpu.VMEM((128, 64), f32),
            pltpu.VMEM((88, 64), f32),
            pltpu.VMEM_SHARED((NR, 64), f32),
            pltpu.SemaphoreType.DMA((DEPTH,)),
            pltpu.SemaphoreType.DMA((DEPTH,)),
        ],
    )(src3, dst3, glo, ghi, z64)


def _agg2_body(srce, dste, q, z48, out, idxg, idxd, rows, st128, st120,
               acc, sem_g, sem_s):
    cid = lax.axis_index("c")
    sid = lax.axis_index("s")
    pltpu.sync_copy(z48, st128)
    pltpu.sync_copy(z48.at[pl.ds(0, 120)], st120)
    r0 = sid * STRIDE2
    for k in range(4):
        pltpu.sync_copy(st128, acc.at[pl.ds(r0 + k * 128, 128)])
    pltpu.sync_copy(st120, acc.at[pl.ds(r0 + 512, 120)])
    plsc.subcore_barrier()
    base = cid * 1280 + sid * 80
    _row_pipeline(srce, dste, q, acc, idxg, idxd, rows, sem_g, sem_s, base, 2)
    plsc.subcore_barrier()
    for k in range(4):
        pltpu.sync_copy(acc.at[pl.ds(r0 + k * 128, 128)], st128)
        pltpu.sync_copy(st128, out.at[cid, pl.ds(r0 + k * 128, 128)])
    pltpu.sync_copy(acc.at[pl.ds(r0 + 512, 120)], st120)
    pltpu.sync_copy(st120, out.at[cid, pl.ds(r0 + 512, 120)])


def _sc_agg2(srce, dste, q, z48):
    return pl.kernel(
        _agg2_body,
        out_type=jax.ShapeDtypeStruct((2, NR2, 48), f32),
        mesh=_mesh,
        scratch_types=[
            pltpu.VMEM((GROUP, 128), i32),
            pltpu.VMEM((GROUP, 128), i32),
            pltpu.VMEM((DEPTH, 128, 48), f32),
            pltpu.VMEM((128, 48), f32),
            pltpu.VMEM((120, 48), f32),
            pltpu.VMEM_SHARED((NR2, 48), f32),
            pltpu.SemaphoreType.DMA((DEPTH,)),
            pltpu.SemaphoreType.DMA((DEPTH,)),
        ],
    )(srce, dste, q, z48)


def kernel(x, edge_index, is_reversed, W_st1, b_st1, W_ts1, b_ts1, W_1, b_1,
           W_2, b_2):
    src = edge_index[0].astype(i32)
    dst = edge_index[1].astype(i32)
    rev = is_reversed.astype(i32)

    s3a, d3a, s3b, d3b = _tc_indices(
        src.reshape(2500, 128), dst.reshape(2500, 128), rev.reshape(2500, 128))
    ar1 = jnp.arange(LEN1 - 2 * E, dtype=i32)
    src3 = jnp.concatenate(
        [s3a.ravel(), s3b.ravel(), ar1 % 64]).reshape(LEN1 // 128, 128)
    dst3 = jnp.concatenate(
        [d3a.ravel(), d3b.ravel(), 3 * N + ar1 % 64]).reshape(LEN1 // 128, 128)
    ar2 = jnp.arange(LEN2 - E, dtype=i32)
    srce = jnp.concatenate([src, ar2 % 64]).reshape(LEN2 // 128, 128)
    dste = jnp.concatenate([dst, N + ar2 % 64]).reshape(LEN2 // 128, 128)

    ones128 = jnp.ones((128,), f32)
    z1880 = jnp.zeros((STRIDE1,), f32)
    z64 = jnp.zeros((128, 64), f32)
    z48 = jnp.zeros((128, 48), f32)

    wcat = jnp.concatenate([W_st1, W_1, W_ts1], axis=1)
    h = _tc_matmul(x, wcat)
    h3p = jnp.pad(h.reshape(3 * N, 128), ((0, NR - 3 * N), (0, 0)))

    hist = _sc_hist(dst3, ones128, z1880)
    glo, ghi, dis = _tc_scale(hist.reshape(2, NR, 1), h3p)

    agg = _sc_agg1(src3, dst3, glo, ghi, z64)

    pad3 = ((0, NR2 - N), (0, 0), (0, 0))
    agglo3 = jnp.pad(agg[0, :3 * N].reshape(N, 3, 64), pad3)
    agghi3 = jnp.pad(agg[1, :3 * N].reshape(N, 3, 64), pad3)
    h33 = jnp.pad(h3p[:3 * N].reshape(N, 3, 128), pad3)
    dis3 = jnp.pad(dis[:3 * N].reshape(N, 3, 1), pad3)
    bcat = jnp.stack([b_st1, b_1, b_ts1])
    w2p = jnp.pad(
        jnp.concatenate([W_2[0:128], W_2[256:384], W_2[128:256]], axis=0),
        ((0, 0), (0, 8))).reshape(3, 128, 48)

    p, q = _tc_l2(agglo3, agghi3, h33, dis3, bcat, w2p)

    parts = _sc_agg2(srce, dste, q, z48)

    out = _tc_out(parts, p, dis3[:, 1, :], b_2.reshape(1, 40))
    return out[:N]


# speedup vs baseline: 18.9468x; 18.9468x over previous
import functools

import jax
import jax.numpy as jnp
from jax import lax
from jax.experimental import pallas as pl
from jax.experimental.pallas import tpu as pltpu
from jax.experimental.pallas import tpu_sc as plsc

f32 = jnp.float32
i32 = jnp.int32

N = 10000
E = 320000
NR = 30080
NR2 = 10112
LEN1 = 655360
LEN2 = 327680
GROUP = 40
DEPTH = 4
STRIDE1 = NR // 16
STRIDE2 = NR2 // 16

def _mesh():
    return plsc.VectorSubcoreMesh(
        core_axis_name="c", subcore_axis_name="s", num_cores=2,
        num_subcores=16)


def _mm_body(xb, wb, ob):
    ob[...] = jnp.dot(xb[...], wb[...], preferred_element_type=f32)


def _tc_matmul(x, wcat):
    return pl.pallas_call(
        _mm_body,
        grid=(10,),
        in_specs=[pl.BlockSpec((1000, 128), lambda i: (i, 0)),
                  pl.BlockSpec((128, 384), lambda i: (0, 0))],
        out_specs=pl.BlockSpec((1000, 384), lambda i: (i, 0)),
        out_shape=jax.ShapeDtypeStruct((N, 384), f32),
    )(x, wcat)


def _idx_body(sb, db, rb, asb, adb, bsb, bdb):
    sv, dv, rv = sb[...], db[...], rb[...]
    asb[...] = sv * 3 + 2 * rv
    adb[...] = dv * 3 + 2 * rv
    bsb[...] = sv * 3 + 1
    bdb[...] = dv * 3 + 1


def _tc_indices(srcm, dstm, revm):
    spec = pl.BlockSpec((2500, 128), lambda i: (0, 0))
    sh = jax.ShapeDtypeStruct((2500, 128), i32)
    return pl.pallas_call(
        _idx_body,
        grid=(1,),
        in_specs=[spec, spec, spec],
        out_specs=[spec, spec, spec, spec],
        out_shape=[sh, sh, sh, sh],
    )(srcm, dstm, revm)


def _scale_body(cb, hb, g0b, g1b, g2b, g3b, db):
    cnt = cb[0] + cb[1]
    dis = lax.rsqrt(cnt + 1.0)
    db[...] = dis
    g = hb[...] * dis
    g0b[...] = g[:, 0:32]
    g1b[...] = g[:, 32:64]
    g2b[...] = g[:, 64:96]
    g3b[...] = g[:, 96:128]


def _tc_scale(cnt, h3p):
    return pl.pallas_call(
        _scale_body,
        grid=(NR // 128,),
        in_specs=[pl.BlockSpec((2, 128, 1), lambda i: (0, i, 0)),
                  pl.BlockSpec((128, 128), lambda i: (i, 0))],
        out_specs=[pl.BlockSpec((128, 32), lambda i: (i, 0))] * 4
        + [pl.BlockSpec((128, 1), lambda i: (i, 0))],
        out_shape=[jax.ShapeDtypeStruct((NR, 32), f32)] * 4
        + [jax.ShapeDtypeStruct((NR, 1), f32)],
    )(cnt, h3p)


def _l2_body(a0b, a1b, a2b, a3b, hb, db, bcb, wb, pb, qb):
    acc = jnp.zeros((128, 48), f32)
    for c in range(3):
        a = jnp.concatenate(
            [a0b[:, c, :], a1b[:, c, :], a2b[:, c, :], a3b[:, c, :]], axis=1)
        dd = db[:, c, :]
        hc = jnp.maximum(dd * a + dd * dd * hb[:, c, :] + bcb[c][None, :], 0.0)
        acc = acc + jnp.dot(hc, wb[c], preferred_element_type=f32)
    pb[...] = acc
    qb[...] = db[:, 1, :] * acc


def _tc_l2(agg3s, h33, dis3, bcat, w2p):
    sh48 = jax.ShapeDtypeStruct((NR2, 48), f32)
    return pl.pallas_call(
        _l2_body,
        grid=(NR2 // 128,),
        in_specs=[pl.BlockSpec((128, 3, 32), lambda i: (i, 0, 0))] * 4
        + [pl.BlockSpec((128, 3, 128), lambda i: (i, 0, 0)),
           pl.BlockSpec((128, 3, 1), lambda i: (i, 0, 0)),
           pl.BlockSpec((3, 128), lambda i: (0, 0)),
           pl.BlockSpec((3, 128, 48), lambda i: (0, 0, 0))],
        out_specs=[pl.BlockSpec((128, 48), lambda i: (i, 0)),
                   pl.BlockSpec((128, 48), lambda i: (i, 0))],
        out_shape=[sh48, sh48],
    )(*agg3s, h33, dis3, bcat, w2p)


def _out_body(ptb, pb, db, b2b, ob):
    s = ptb[0] + ptb[1]
    dd = db[...]
    o = dd * s[:, :40] + dd * dd * pb[:, :40] + b2b[...]
    m = jnp.max(o, axis=1, keepdims=True)
    z = jnp.sum(jnp.exp(o - m), axis=1, keepdims=True)
    ob[...] = o - m - jnp.log(z)


def _tc_out(parts, p, disall, b2p):
    return pl.pallas_call(
        _out_body,
        grid=(NR2 // 128,),
        in_specs=[pl.BlockSpec((2, 128, 48), lambda i: (0, i, 0)),
                  pl.BlockSpec((128, 48), lambda i: (i, 0)),
                  pl.BlockSpec((128, 1), lambda i: (i, 0)),
                  pl.BlockSpec((1, 40), lambda i: (0, 0))],
        out_specs=pl.BlockSpec((128, 40), lambda i: (i, 0)),
        out_shape=jax.ShapeDtypeStruct((NR2, 40), f32),
    )(parts, p, disall, b2p)


def _hist_body(dst3, ones_h, z_h, out, idxd, ones_v, stage, hist, sem):
    cid = lax.axis_index("c")
    sid = lax.axis_index("s")
    pltpu.sync_copy(z_h, stage)
    pltpu.sync_copy(stage, hist.at[pl.ds(sid * STRIDE1, STRIDE1)])
    pltpu.sync_copy(ones_h, ones_v)
    plsc.subcore_barrier()
    base = (cid * 16 + sid) * 160
    for g in range(4):
        pltpu.sync_copy(dst3.at[pl.ds(base + g * GROUP, GROUP)], idxd)

        def it_body(it, carry):
            for b in range(8):
                pltpu.async_copy(ones_v, hist.at[idxd.at[it * 8 + b]], sem,
                                 add=True)
            for b in range(8):
                pltpu.make_async_copy(ones_v, hist.at[idxd.at[0]], sem).wait()
            return carry

        lax.fori_loop(0, 5, it_body, 0)
    plsc.subcore_barrier()
    pltpu.sync_copy(hist.at[pl.ds(sid * STRIDE1, STRIDE1)], stage)
    pltpu.sync_copy(stage, out.at[cid, sid])


def _sc_hist(dst3, ones_h, z_h):
    return pl.kernel(
        _hist_body,
        out_type=jax.ShapeDtypeStruct((2, 16, STRIDE1), f32),
        mesh=_mesh(),
        compiler_params=pltpu.CompilerParams(use_tc_tiling_on_sc=False),
        scratch_types=[
            pltpu.VMEM((GROUP, 128), i32),
            pltpu.VMEM((128,), f32),
            pltpu.VMEM((STRIDE1,), f32),
            pltpu.VMEM_SHARED((NR,), f32),
            pltpu.SemaphoreType.DMA,
        ],
    )(dst3, ones_h, z_h)


def _row_pipeline(src_hbm, dst_hbm, table, acc, idxg, idxd, rows, sem_g,
                  sem_s, base, groups):
    for g in range(groups):
        if g > 0:
            for b in range(DEPTH):
                pltpu.make_async_copy(
                    rows.at[b], acc.at[idxd.at[0]], sem_s.at[b]).wait()
        pltpu.sync_copy(src_hbm.at[pl.ds(base + g * GROUP, GROUP)], idxg)
        pltpu.sync_copy(dst_hbm.at[pl.ds(base + g * GROUP, GROUP)], idxd)

        def it_body(it, carry):
            for b in range(DEPTH):
                @pl.when(it > 0)
                def _drain(b=b):
                    pltpu.make_async_copy(
                        rows.at[b], acc.at[idxd.at[0]], sem_s.at[b]).wait()
                pltpu.async_copy(
                    table.at[idxg.at[it * DEPTH + b]], rows.at[b],
                    sem_g.at[b])
            for b in range(DEPTH):
                pltpu.make_async_copy(
                    table.at[idxg.at[0]], rows.at[b], sem_g.at[b]).wait()
                pltpu.async_copy(
                    rows.at[b], acc.at[idxd.at[it * DEPTH + b]],
                    sem_s.at[b], add=True)
            return carry

        lax.fori_loop(0, GROUP // DEPTH, it_body, 0)
    for b in range(DEPTH):
        pltpu.make_async_copy(rows.at[b], acc.at[idxd.at[0]], sem_s.at[b]).wait()


def _agg1_body(src3, dst3, g0, g1, g2, g3, z32, out, idxg, idxd, rows, st128,
               st88, acc, sem_g, sem_s):
    cid = lax.axis_index("c")
    sid = lax.axis_index("s")
    r0 = sid * STRIDE1
    base = sid * 320
    for p in range(2):
        pltpu.sync_copy(z32, st128)
        pltpu.sync_copy(z32.at[pl.ds(0, 88)], st88)
        for k in range(14):
            pltpu.sync_copy(st128, acc.at[pl.ds(r0 + k * 128, 128)])
        pltpu.sync_copy(st88, acc.at[pl.ds(r0 + 1792, 88)])
        plsc.subcore_barrier()
        pl.when(cid == 0)(lambda p=p: _row_pipeline(
            src3, dst3, (g0, g1)[p], acc, idxg, idxd, rows, sem_g, sem_s,
            base, 8))
        pl.when(cid == 1)(lambda p=p: _row_pipeline(
            src3, dst3, (g2, g3)[p], acc, idxg, idxd, rows, sem_g, sem_s,
            base, 8))
        plsc.subcore_barrier()
        for k in range(14):
            pltpu.sync_copy(acc.at[pl.ds(r0 + k * 128, 128)], st128)
            pltpu.sync_copy(st128, out.at[cid * 2 + p, pl.ds(r0 + k * 128, 128)])
        pltpu.sync_copy(acc.at[pl.ds(r0 + 1792, 88)], st88)
        pltpu.sync_copy(st88, out.at[cid * 2 + p, pl.ds(r0 + 1792, 88)])
        plsc.subcore_barrier()


def _sc_agg1(src3, dst3, g0, g1, g2, g3, z32):
    return pl.kernel(
        _agg1_body,
        out_type=jax.ShapeDtypeStruct((4, NR, 32), f32),
        mesh=_mesh(),
        compiler_params=pltpu.CompilerParams(use_tc_tiling_on_sc=False),
        scratch_types=[
            pltpu.VMEM((GROUP, 128), i32),
            pltpu.VMEM((GROUP, 128), i32),
            pltpu.VMEM((DEPTH, 128, 32), f32),
            pltpu.VMEM((128, 32), f32),
            pltpu.VMEM((88, 32), f32),
            pltpu.VMEM_SHARED((NR, 32), f32),
            pltpu.SemaphoreType.DMA((DEPTH,)),
            pltpu.SemaphoreType.DMA((DEPTH,)),
        ],
    )(src3, dst3, g0, g1, g2, g3, z32)


def _agg2_body(srce, dste, q, z48, out, idxg, idxd, rows, st128, st120,
               acc, sem_g, sem_s):
    cid = lax.axis_index("c")
    sid = lax.axis_index("s")
    pltpu.sync_copy(z48, st128)
    pltpu.sync_copy(z48.at[pl.ds(0, 120)], st120)
    r0 = sid * STRIDE2
    for k in range(4):
        pltpu.sync_copy(st128, acc.at[pl.ds(r0 + k * 128, 128)])
    pltpu.sync_copy(st120, acc.at[pl.ds(r0 + 512, 120)])
    plsc.subcore_barrier()
    base = cid * 1280 + sid * 80
    _row_pipeline(srce, dste, q, acc, idxg, idxd, rows, sem_g, sem_s, base, 2)
    plsc.subcore_barrier()
    for k in range(4):
        pltpu.sync_copy(acc.at[pl.ds(r0 + k * 128, 128)], st128)
        pltpu.sync_copy(st128, out.at[cid, pl.ds(r0 + k * 128, 128)])
    pltpu.sync_copy(acc.at[pl.ds(r0 + 512, 120)], st120)
    pltpu.sync_copy(st120, out.at[cid, pl.ds(r0 + 512, 120)])


def _sc_agg2(srce, dste, q, z48):
    return pl.kernel(
        _agg2_body,
        out_type=jax.ShapeDtypeStruct((2, NR2, 48), f32),
        mesh=_mesh(),
        compiler_params=pltpu.CompilerParams(use_tc_tiling_on_sc=False),
        scratch_types=[
            pltpu.VMEM((GROUP, 128), i32),
            pltpu.VMEM((GROUP, 128), i32),
            pltpu.VMEM((DEPTH, 128, 48), f32),
            pltpu.VMEM((128, 48), f32),
            pltpu.VMEM((120, 48), f32),
            pltpu.VMEM_SHARED((NR2, 48), f32),
            pltpu.SemaphoreType.DMA((DEPTH,)),
            pltpu.SemaphoreType.DMA((DEPTH,)),
        ],
    )(srce, dste, q, z48)


def kernel(x, edge_index, is_reversed, W_st1, b_st1, W_ts1, b_ts1, W_1, b_1,
           W_2, b_2):
    src = edge_index[0].astype(i32)
    dst = edge_index[1].astype(i32)
    rev = is_reversed.astype(i32)

    s3a, d3a, s3b, d3b = _tc_indices(
        src.reshape(2500, 128), dst.reshape(2500, 128), rev.reshape(2500, 128))
    ar1 = jnp.arange(LEN1 - 2 * E, dtype=i32)
    src3 = jnp.concatenate(
        [s3a.ravel(), s3b.ravel(), ar1 % 64]).reshape(LEN1 // 128, 128)
    dst3 = jnp.concatenate(
        [d3a.ravel(), d3b.ravel(), 3 * N + ar1 % 64]).reshape(LEN1 // 128, 128)
    ar2 = jnp.arange(LEN2 - E, dtype=i32)
    srce = jnp.concatenate([src, ar2 % 64]).reshape(LEN2 // 128, 128)
    dste = jnp.concatenate([dst, N + ar2 % 64]).reshape(LEN2 // 128, 128)

    ones128 = jnp.ones((128,), f32)
    z1880 = jnp.zeros((STRIDE1,), f32)
    z32 = jnp.zeros((128, 32), f32)
    z48 = jnp.zeros((128, 48), f32)

    wcat = jnp.concatenate([W_st1, W_1, W_ts1], axis=1)
    h = _tc_matmul(x, wcat)
    h3p = jnp.pad(h.reshape(3 * N, 128), ((0, NR - 3 * N), (0, 0)))

    hist = _sc_hist(dst3, ones128, z1880)
    g0, g1, g2, g3, dis = _tc_scale(hist.reshape(2, NR, 1), h3p)

    agg = _sc_agg1(src3, dst3, g0, g1, g2, g3, z32)

    pad3 = ((0, NR2 - N), (0, 0), (0, 0))
    agg3s = [jnp.pad(agg[k, :3 * N].reshape(N, 3, 32), pad3)
             for k in range(4)]
    h33 = jnp.pad(h3p[:3 * N].reshape(N, 3, 128), pad3)
    dis3 = jnp.pad(dis[:3 * N].reshape(N, 3, 1), pad3)
    bcat = jnp.stack([b_st1, b_1, b_ts1])
    w2p = jnp.pad(
        jnp.concatenate([W_2[0:128], W_2[256:384], W_2[128:256]], axis=0),
        ((0, 0), (0, 8))).reshape(3, 128, 48)

    p, q = _tc_l2(agg3s, h33, dis3, bcat, w2p)

    parts = _sc_agg2(srce, dste, q, z48)

    out = _tc_out(parts, p, dis3[:, 1, :], b_2.reshape(1, 40))
    return out[:N]


# speedup vs baseline: 19.6611x; 1.0377x over previous
import functools

import jax
import jax.numpy as jnp
from jax import lax
from jax.experimental import pallas as pl
from jax.experimental.pallas import tpu as pltpu
from jax.experimental.pallas import tpu_sc as plsc

f32 = jnp.float32
i32 = jnp.int32

N = 10000
E = 320000
NR = 30080
NR2 = 10112
LEN1 = 655360
LEN2 = 327680
GROUP1 = 80
GROUP2 = 40
DEPTH = 8
STRIDE1 = NR // 16
STRIDE2 = NR2 // 16

def _mesh():
    return plsc.VectorSubcoreMesh(
        core_axis_name="c", subcore_axis_name="s", num_cores=2,
        num_subcores=16)


def _mm_body(xb, wb, ob):
    ob[...] = jnp.dot(xb[...], wb[...], preferred_element_type=f32)


def _tc_matmul(x, wcat):
    return pl.pallas_call(
        _mm_body,
        grid=(10,),
        in_specs=[pl.BlockSpec((1000, 128), lambda i: (i, 0)),
                  pl.BlockSpec((128, 384), lambda i: (0, 0))],
        out_specs=pl.BlockSpec((1000, 384), lambda i: (i, 0)),
        out_shape=jax.ShapeDtypeStruct((N, 384), f32),
    )(x, wcat)


def _idx_body(sb, db, rb, asb, adb, bsb, bdb):
    sv, dv, rv = sb[...], db[...], rb[...]
    asb[...] = sv * 3 + 2 * rv
    adb[...] = dv * 3 + 2 * rv
    bsb[...] = sv * 3 + 1
    bdb[...] = dv * 3 + 1


def _tc_indices(srcm, dstm, revm):
    spec = pl.BlockSpec((2500, 128), lambda i: (0, 0))
    sh = jax.ShapeDtypeStruct((2500, 128), i32)
    return pl.pallas_call(
        _idx_body,
        grid=(1,),
        in_specs=[spec, spec, spec],
        out_specs=[spec, spec, spec, spec],
        out_shape=[sh, sh, sh, sh],
    )(srcm, dstm, revm)


def _scale_body(cb, hb, g0b, g1b, g2b, g3b, db):
    cnt = cb[0] + cb[1]
    dis = lax.rsqrt(cnt + 1.0)
    db[...] = dis
    g = hb[...] * dis
    g0b[...] = g[:, 0:32]
    g1b[...] = g[:, 32:64]
    g2b[...] = g[:, 64:96]
    g3b[...] = g[:, 96:128]


def _tc_scale(cnt, h3p):
    return pl.pallas_call(
        _scale_body,
        grid=(NR // 128,),
        in_specs=[pl.BlockSpec((2, 128, 1), lambda i: (0, i, 0)),
                  pl.BlockSpec((128, 128), lambda i: (i, 0))],
        out_specs=[pl.BlockSpec((128, 32), lambda i: (i, 0))] * 4
        + [pl.BlockSpec((128, 1), lambda i: (i, 0))],
        out_shape=[jax.ShapeDtypeStruct((NR, 32), f32)] * 4
        + [jax.ShapeDtypeStruct((NR, 1), f32)],
    )(cnt, h3p)


def _l2_body(a0b, a1b, a2b, a3b, hb, db, bcb, wb, pb, qb):
    acc = jnp.zeros((128, 48), f32)
    for c in range(3):
        a = jnp.concatenate(
            [a0b[:, c, :], a1b[:, c, :], a2b[:, c, :], a3b[:, c, :]], axis=1)
        dd = db[:, c, :]
        hc = jnp.maximum(dd * a + dd * dd * hb[:, c, :] + bcb[c][None, :], 0.0)
        acc = acc + jnp.dot(hc, wb[c], preferred_element_type=f32)
    pb[...] = acc
    qb[...] = db[:, 1, :] * acc


def _tc_l2(agg3s, h33, dis3, bcat, w2p):
    sh48 = jax.ShapeDtypeStruct((NR2, 48), f32)
    return pl.pallas_call(
        _l2_body,
        grid=(NR2 // 128,),
        in_specs=[pl.BlockSpec((128, 3, 32), lambda i: (i, 0, 0))] * 4
        + [pl.BlockSpec((128, 3, 128), lambda i: (i, 0, 0)),
           pl.BlockSpec((128, 3, 1), lambda i: (i, 0, 0)),
           pl.BlockSpec((3, 128), lambda i: (0, 0)),
           pl.BlockSpec((3, 128, 48), lambda i: (0, 0, 0))],
        out_specs=[pl.BlockSpec((128, 48), lambda i: (i, 0)),
                   pl.BlockSpec((128, 48), lambda i: (i, 0))],
        out_shape=[sh48, sh48],
    )(*agg3s, h33, dis3, bcat, w2p)


def _out_body(ptb, pb, db, b2b, ob):
    s = ptb[0] + ptb[1]
    dd = db[...]
    o = dd * s[:, :40] + dd * dd * pb[:, :40] + b2b[...]
    m = jnp.max(o, axis=1, keepdims=True)
    z = jnp.sum(jnp.exp(o - m), axis=1, keepdims=True)
    ob[...] = o - m - jnp.log(z)


def _tc_out(parts, p, disall, b2p):
    return pl.pallas_call(
        _out_body,
        grid=(NR2 // 128,),
        in_specs=[pl.BlockSpec((2, 128, 48), lambda i: (0, i, 0)),
                  pl.BlockSpec((128, 48), lambda i: (i, 0)),
                  pl.BlockSpec((128, 1), lambda i: (i, 0)),
                  pl.BlockSpec((1, 40), lambda i: (0, 0))],
        out_specs=pl.BlockSpec((128, 40), lambda i: (i, 0)),
        out_shape=jax.ShapeDtypeStruct((NR2, 40), f32),
    )(parts, p, disall, b2p)


def _hist_body(dst3, ones_h, z_h, out, idxd, ones_v, stage, hist, sem):
    cid = lax.axis_index("c")
    sid = lax.axis_index("s")
    pltpu.sync_copy(z_h, stage)
    pltpu.sync_copy(stage, hist.at[pl.ds(sid * STRIDE1, STRIDE1)])
    pltpu.sync_copy(ones_h, ones_v)
    plsc.subcore_barrier()
    base = (cid * 16 + sid) * 160
    for g in range(4):
        pltpu.sync_copy(dst3.at[pl.ds(base + g * GROUP2, GROUP2)], idxd)

        def it_body(it, carry):
            for b in range(8):
                pltpu.async_copy(ones_v, hist.at[idxd.at[it * 8 + b]], sem,
                                 add=True)
            for b in range(8):
                pltpu.make_async_copy(ones_v, hist.at[idxd.at[0]], sem).wait()
            return carry

        lax.fori_loop(0, 5, it_body, 0)
    plsc.subcore_barrier()
    pltpu.sync_copy(hist.at[pl.ds(sid * STRIDE1, STRIDE1)], stage)
    pltpu.sync_copy(stage, out.at[cid, sid])


def _sc_hist(dst3, ones_h, z_h):
    return pl.kernel(
        _hist_body,
        out_type=jax.ShapeDtypeStruct((2, 16, STRIDE1), f32),
        mesh=_mesh(),
        compiler_params=pltpu.CompilerParams(use_tc_tiling_on_sc=False),
        scratch_types=[
            pltpu.VMEM((GROUP2, 128), i32),
            pltpu.VMEM((128,), f32),
            pltpu.VMEM((STRIDE1,), f32),
            pltpu.VMEM_SHARED((NR,), f32),
            pltpu.SemaphoreType.DMA,
        ],
    )(dst3, ones_h, z_h)


def _row_pipeline(src_hbm, dst_hbm, table, acc, idxg, idxd, rows, sem_g,
                  sem_s, base, groups, group):
    for g in range(groups):
        if g > 0:
            for b in range(DEPTH):
                pltpu.make_async_copy(
                    rows.at[b], acc.at[idxd.at[0]], sem_s.at[b]).wait()
        pltpu.sync_copy(src_hbm.at[pl.ds(base + g * group, group)], idxg)
        pltpu.sync_copy(dst_hbm.at[pl.ds(base + g * group, group)], idxd)

        def it_body(it, carry):
            for b in range(DEPTH):
                @pl.when(it > 0)
                def _drain(b=b):
                    pltpu.make_async_copy(
                        rows.at[b], acc.at[idxd.at[0]], sem_s.at[b]).wait()
                pltpu.async_copy(
                    table.at[idxg.at[it * DEPTH + b]], rows.at[b],
                    sem_g.at[b])
            for b in range(DEPTH):
                pltpu.make_async_copy(
                    table.at[idxg.at[0]], rows.at[b], sem_g.at[b]).wait()
                pltpu.async_copy(
                    rows.at[b], acc.at[idxd.at[it * DEPTH + b]],
                    sem_s.at[b], add=True)
            return carry

        lax.fori_loop(0, group // DEPTH, it_body, 0)
    for b in range(DEPTH):
        pltpu.make_async_copy(rows.at[b], acc.at[idxd.at[0]], sem_s.at[b]).wait()


def _agg1_body(src3, dst3, g0, g1, g2, g3, z32, out, idxg, idxd, rows, st128,
               st88, acc, sem_g, sem_s):
    cid = lax.axis_index("c")
    sid = lax.axis_index("s")
    r0 = sid * STRIDE1
    base = sid * 320
    for p in range(2):
        pltpu.sync_copy(z32, st128)
        pltpu.sync_copy(z32.at[pl.ds(0, 88)], st88)
        for k in range(14):
            pltpu.sync_copy(st128, acc.at[pl.ds(r0 + k * 128, 128)])
        pltpu.sync_copy(st88, acc.at[pl.ds(r0 + 1792, 88)])
        plsc.subcore_barrier()
        pl.when(cid == 0)(lambda p=p: _row_pipeline(
            src3, dst3, (g0, g1)[p], acc, idxg, idxd, rows, sem_g, sem_s,
            base, 4, GROUP1))
        pl.when(cid == 1)(lambda p=p: _row_pipeline(
            src3, dst3, (g2, g3)[p], acc, idxg, idxd, rows, sem_g, sem_s,
            base, 4, GROUP1))
        plsc.subcore_barrier()
        for k in range(14):
            pltpu.sync_copy(acc.at[pl.ds(r0 + k * 128, 128)], st128)
            pltpu.sync_copy(st128, out.at[cid * 2 + p, pl.ds(r0 + k * 128, 128)])
        pltpu.sync_copy(acc.at[pl.ds(r0 + 1792, 88)], st88)
        pltpu.sync_copy(st88, out.at[cid * 2 + p, pl.ds(r0 + 1792, 88)])
        plsc.subcore_barrier()


def _sc_agg1(src3, dst3, g0, g1, g2, g3, z32):
    return pl.kernel(
        _agg1_body,
        out_type=jax.ShapeDtypeStruct((4, NR, 32), f32),
        mesh=_mesh(),
        compiler_params=pltpu.CompilerParams(use_tc_tiling_on_sc=False),
        scratch_types=[
            pltpu.VMEM((GROUP1, 128), i32),
            pltpu.VMEM((GROUP1, 128), i32),
            pltpu.VMEM((DEPTH, 128, 32), f32),
            pltpu.VMEM((128, 32), f32),
            pltpu.VMEM((88, 32), f32),
            pltpu.VMEM_SHARED((NR, 32), f32),
            pltpu.SemaphoreType.DMA((DEPTH,)),
            pltpu.SemaphoreType.DMA((DEPTH,)),
        ],
    )(src3, dst3, g0, g1, g2, g3, z32)


def _agg2_body(srce, dste, q, z48, out, idxg, idxd, rows, st128, st120,
               acc, sem_g, sem_s):
    cid = lax.axis_index("c")
    sid = lax.axis_index("s")
    pltpu.sync_copy(z48, st128)
    pltpu.sync_copy(z48.at[pl.ds(0, 120)], st120)
    r0 = sid * STRIDE2
    for k in range(4):
        pltpu.sync_copy(st128, acc.at[pl.ds(r0 + k * 128, 128)])
    pltpu.sync_copy(st120, acc.at[pl.ds(r0 + 512, 120)])
    plsc.subcore_barrier()
    base = cid * 1280 + sid * 80
    _row_pipeline(srce, dste, q, acc, idxg, idxd, rows, sem_g, sem_s, base, 2,
                  GROUP2)
    plsc.subcore_barrier()
    for k in range(4):
        pltpu.sync_copy(acc.at[pl.ds(r0 + k * 128, 128)], st128)
        pltpu.sync_copy(st128, out.at[cid, pl.ds(r0 + k * 128, 128)])
    pltpu.sync_copy(acc.at[pl.ds(r0 + 512, 120)], st120)
    pltpu.sync_copy(st120, out.at[cid, pl.ds(r0 + 512, 120)])


def _sc_agg2(srce, dste, q, z48):
    return pl.kernel(
        _agg2_body,
        out_type=jax.ShapeDtypeStruct((2, NR2, 48), f32),
        mesh=_mesh(),
        compiler_params=pltpu.CompilerParams(use_tc_tiling_on_sc=False),
        scratch_types=[
            pltpu.VMEM((GROUP2, 128), i32),
            pltpu.VMEM((GROUP2, 128), i32),
            pltpu.VMEM((DEPTH, 128, 48), f32),
            pltpu.VMEM((128, 48), f32),
            pltpu.VMEM((120, 48), f32),
            pltpu.VMEM_SHARED((NR2, 48), f32),
            pltpu.SemaphoreType.DMA((DEPTH,)),
            pltpu.SemaphoreType.DMA((DEPTH,)),
        ],
    )(srce, dste, q, z48)


def kernel(x, edge_index, is_reversed, W_st1, b_st1, W_ts1, b_ts1, W_1, b_1,
           W_2, b_2):
    src = edge_index[0].astype(i32)
    dst = edge_index[1].astype(i32)
    rev = is_reversed.astype(i32)

    s3a, d3a, s3b, d3b = _tc_indices(
        src.reshape(2500, 128), dst.reshape(2500, 128), rev.reshape(2500, 128))
    ar1 = jnp.arange(LEN1 - 2 * E, dtype=i32)
    src3 = jnp.concatenate(
        [s3a.ravel(), s3b.ravel(), ar1 % 64]).reshape(LEN1 // 128, 128)
    dst3 = jnp.concatenate(
        [d3a.ravel(), d3b.ravel(), 3 * N + ar1 % 64]).reshape(LEN1 // 128, 128)
    ar2 = jnp.arange(LEN2 - E, dtype=i32)
    srce = jnp.concatenate([src, ar2 % 64]).reshape(LEN2 // 128, 128)
    dste = jnp.concatenate([dst, N + ar2 % 64]).reshape(LEN2 // 128, 128)

    ones128 = jnp.ones((128,), f32)
    z1880 = jnp.zeros((STRIDE1,), f32)
    z32 = jnp.zeros((128, 32), f32)
    z48 = jnp.zeros((128, 48), f32)

    wcat = jnp.concatenate([W_st1, W_1, W_ts1], axis=1)
    h = _tc_matmul(x, wcat)
    h3p = jnp.pad(h.reshape(3 * N, 128), ((0, NR - 3 * N), (0, 0)))

    hist = _sc_hist(dst3, ones128, z1880)
    g0, g1, g2, g3, dis = _tc_scale(hist.reshape(2, NR, 1), h3p)

    agg = _sc_agg1(src3, dst3, g0, g1, g2, g3, z32)

    pad3 = ((0, NR2 - N), (0, 0), (0, 0))
    agg3s = [jnp.pad(agg[k, :3 * N].reshape(N, 3, 32), pad3)
             for k in range(4)]
    h33 = jnp.pad(h3p[:3 * N].reshape(N, 3, 128), pad3)
    dis3 = jnp.pad(dis[:3 * N].reshape(N, 3, 1), pad3)
    bcat = jnp.stack([b_st1, b_1, b_ts1])
    w2p = jnp.pad(
        jnp.concatenate([W_2[0:128], W_2[256:384], W_2[128:256]], axis=0),
        ((0, 0), (0, 8))).reshape(3, 128, 48)

    p, q = _tc_l2(agg3s, h33, dis3, bcat, w2p)

    parts = _sc_agg2(srce, dste, q, z48)

    out = _tc_out(parts, p, dis3[:, 1, :], b_2.reshape(1, 40))
    return out[:N]


# speedup vs baseline: 22.8688x; 1.1631x over previous
import jax
import jax.numpy as jnp
from jax import lax
from jax.experimental import pallas as pl
from jax.experimental.pallas import tpu as pltpu
from jax.experimental.pallas import tpu_sc as plsc

f32 = jnp.float32
i32 = jnp.int32

N = 10000
E = 320000
NP = 10112
N3 = 3 * NP
NRH = 30080
NE2 = 2 * NP
LENH = 655360
LENE = 327680
GROUP = 40
DEPTH1 = 4
DEPTH2 = 8
STRH = NRH // 16
STR1 = NE2 // 16
STR2 = NP // 16


def _mesh():
    return plsc.VectorSubcoreMesh(
        core_axis_name="c", subcore_axis_name="s", num_cores=2,
        num_subcores=16)


_SC_PARAMS = dict(compiler_params=pltpu.CompilerParams(
    use_tc_tiling_on_sc=False))


def _mm_body(xb, wb, ob):
    ob[...] = jnp.dot(xb[...], wb[...], preferred_element_type=f32)


def _tc_matmul(xp, wcat):
    return pl.pallas_call(
        _mm_body,
        grid=(NP // 128,),
        in_specs=[pl.BlockSpec((128, 128), lambda i: (i, 0)),
                  pl.BlockSpec((128, 384), lambda i: (0, 0))],
        out_specs=pl.BlockSpec((128, 384), lambda i: (i, 0)),
        out_shape=jax.ShapeDtypeStruct((NP, 384), f32),
    )(xp, wcat)


def _idx_body(sb, db, rb, da, db_, es, ed):
    sv, dv, rv = sb[...], db[...], rb[...]
    da[...] = dv * 3 + 2 * rv
    db_[...] = dv * 3 + 1
    es[...] = sv * 2 + rv
    ed[...] = dv * 2 + rv


def _tc_indices(srcm, dstm, revm):
    spec = pl.BlockSpec((2500, 128), lambda i: (0, 0))
    sh = jax.ShapeDtypeStruct((2500, 128), i32)
    return pl.pallas_call(
        _idx_body,
        grid=(1,),
        in_specs=[spec, spec, spec],
        out_specs=[spec, spec, spec, spec],
        out_shape=[sh, sh, sh, sh],
    )(srcm, dstm, revm)


def _tab_body(cb, hb, t0, t1, t2, t3, db):
    dis = lax.rsqrt(cb[0] + cb[1] + 1.0)
    db[...] = dis
    gm = dis * hb[...]
    gst, gall, gts = gm[:, 0, :], gm[:, 1, :], gm[:, 2, :]
    for g, tr in enumerate((t0, t1, t2, t3)):
        cg = slice(32 * g, 32 * g + 32)
        ra = jnp.concatenate([gst[:, cg], gall[:, cg]], axis=1)[:, None, :]
        rb = jnp.concatenate([gts[:, cg], gall[:, cg]], axis=1)[:, None, :]
        tr[...] = jnp.concatenate([ra, rb], axis=1)


def _tc_tables(cnt3, h33):
    sht = jax.ShapeDtypeStruct((NP, 2, 64), f32)
    return pl.pallas_call(
        _tab_body,
        grid=(NP // 128,),
        in_specs=[pl.BlockSpec((2, 128, 3, 1), lambda i: (0, i, 0, 0)),
                  pl.BlockSpec((128, 3, 128), lambda i: (i, 0, 0))],
        out_specs=[pl.BlockSpec((128, 2, 64), lambda i: (i, 0, 0))] * 4
        + [pl.BlockSpec((128, 3, 1), lambda i: (i, 0, 0))],
        out_shape=[sht] * 4 + [jax.ShapeDtypeStruct((NP, 3, 1), f32)],
    )(cnt3, h33)


def _l2_body(m00, m01, m02, m03, m10, m11, m12, m13, hb, db, bcb, wb, pb, qb):
    m0s, m1s = (m00, m01, m02, m03), (m10, m11, m12, m13)
    st, ts, al = [], [], []
    for g in range(4):
        sp = m0s[g][...] + m1s[g][...]
        st.append(sp[:, 0, 0:32])
        ts.append(sp[:, 1, 0:32])
        al.append(sp[:, 0, 32:64] + sp[:, 1, 32:64])
    aggs = (jnp.concatenate(st, axis=1), jnp.concatenate(al, axis=1),
            jnp.concatenate(ts, axis=1))
    acc = jnp.zeros((128, 48), f32)
    for c in range(3):
        dd = db[:, c, :]
        hc = jnp.maximum(
            dd * aggs[c] + dd * dd * hb[:, c, :] + bcb[c][None, :], 0.0)
        acc = acc + jnp.dot(hc, wb[c], preferred_element_type=f32)
    pb[...] = acc
    qb[...] = db[:, 1, :] * acc


def _tc_l2(parts1, h33, dis3, bcat, w2p):
    sh48 = jax.ShapeDtypeStruct((NP, 48), f32)
    return pl.pallas_call(
        _l2_body,
        grid=(NP // 128,),
        in_specs=[pl.BlockSpec((128, 2, 64), lambda i: (i, 0, 0))] * 8
        + [pl.BlockSpec((128, 3, 128), lambda i: (i, 0, 0)),
           pl.BlockSpec((128, 3, 1), lambda i: (i, 0, 0)),
           pl.BlockSpec((3, 128), lambda i: (0, 0)),
           pl.BlockSpec((3, 128, 48), lambda i: (0, 0, 0))],
        out_specs=[pl.BlockSpec((128, 48), lambda i: (i, 0)),
                   pl.BlockSpec((128, 48), lambda i: (i, 0))],
        out_shape=[sh48, sh48],
    )(*parts1, h33, dis3, bcat, w2p)


def _out_body(ptb, pb, db, b2b, ob):
    s = ptb[0] + ptb[1]
    dd = db[...]
    o = dd * s[:, :40] + dd * dd * pb[:, :40] + b2b[...]
    m = jnp.max(o, axis=1, keepdims=True)
    z = jnp.sum(jnp.exp(o - m), axis=1, keepdims=True)
    ob[...] = o - m - jnp.log(z)


def _tc_out(parts, p, disall, b2p):
    return pl.pallas_call(
        _out_body,
        grid=(NP // 128,),
        in_specs=[pl.BlockSpec((2, 128, 48), lambda i: (0, i, 0)),
                  pl.BlockSpec((128, 48), lambda i: (i, 0)),
                  pl.BlockSpec((128, 1), lambda i: (i, 0)),
                  pl.BlockSpec((1, 40), lambda i: (0, 0))],
        out_specs=pl.BlockSpec((128, 40), lambda i: (i, 0)),
        out_shape=jax.ShapeDtypeStruct((NP, 40), f32),
    )(parts, p, disall, b2p)


def _hist_body(dst3, ones_h, z_h, out, idxd, ones_v, stage, hist, sem):
    cid = lax.axis_index("c")
    sid = lax.axis_index("s")
    pltpu.sync_copy(z_h, stage)
    pltpu.sync_copy(stage, hist.at[pl.ds(sid * STRH, STRH)])
    pltpu.sync_copy(ones_h, ones_v)
    plsc.subcore_barrier()
    base = (cid * 16 + sid) * 160
    for g in range(4):
        pltpu.sync_copy(dst3.at[pl.ds(base + g * GROUP, GROUP)], idxd)

        def it_body(it, carry):
            for b in range(8):
                pltpu.async_copy(ones_v, hist.at[idxd.at[it * 8 + b]], sem,
                                 add=True)
            for b in range(8):
                pltpu.make_async_copy(ones_v, hist.at[idxd.at[0]], sem).wait()
            return carry

        lax.fori_loop(0, 5, it_body, 0)
    plsc.subcore_barrier()
    pltpu.sync_copy(hist.at[pl.ds(sid * STRH, STRH)], stage)
    pltpu.sync_copy(stage, out.at[cid, sid])


def _sc_hist(dst3, ones_h, z_h):
    return pl.kernel(
        _hist_body,
        out_type=jax.ShapeDtypeStruct((2, 16, STRH), f32),
        mesh=_mesh(),
        scratch_types=[
            pltpu.VMEM((GROUP, 128), i32),
            pltpu.VMEM((128,), f32),
            pltpu.VMEM((STRH,), f32),
            pltpu.VMEM_SHARED((NRH,), f32),
            pltpu.SemaphoreType.DMA,
        ],
        **_SC_PARAMS,
    )(dst3, ones_h, z_h)


def _row_pipeline(src_hbm, dst_hbm, table, acc, idxg, idxd, rows, sem_g,
                  sem_s, base, groups, depth):
    for g in range(groups):
        if g > 0:
            for b in range(depth):
                pltpu.make_async_copy(
                    rows.at[b], acc.at[idxd.at[0]], sem_s.at[b]).wait()
        pltpu.sync_copy(src_hbm.at[pl.ds(base + g * GROUP, GROUP)], idxg)
        pltpu.sync_copy(dst_hbm.at[pl.ds(base + g * GROUP, GROUP)], idxd)

        def it_body(it, carry):
            for b in range(depth):
                @pl.when(it > 0)
                def _drain(b=b):
                    pltpu.make_async_copy(
                        rows.at[b], acc.at[idxd.at[0]], sem_s.at[b]).wait()
                pltpu.async_copy(
                    table.at[idxg.at[it * depth + b]], rows.at[b],
                    sem_g.at[b])
            for b in range(depth):
                pltpu.make_async_copy(
                    table.at[idxg.at[0]], rows.at[b], sem_g.at[b]).wait()
                pltpu.async_copy(
                    rows.at[b], acc.at[idxd.at[it * depth + b]],
                    sem_s.at[b], add=True)
            return carry

        lax.fori_loop(0, GROUP // depth, it_body, 0)
    for b in range(depth):
        pltpu.make_async_copy(rows.at[b], acc.at[idxd.at[0]], sem_s.at[b]).wait()


def _agg1_body(e2s, e2d, t0, t1, t2, t3, z64, out, idxg, idxd, rows, acc,
               sem_g, sem_s):
    cid = lax.axis_index("c")
    sid = lax.axis_index("s")
    r0 = sid * STR1
    base = cid * 1280 + sid * 80
    for p, tbl in enumerate((t0, t1, t2, t3)):
        pltpu.sync_copy(z64, rows.at[0])
        for k in range(9):
            pltpu.sync_copy(rows.at[0], acc.at[pl.ds(r0 + k * 128, 128)])
        pltpu.sync_copy(rows.at[0, pl.ds(0, 112)],
                        acc.at[pl.ds(r0 + 1152, 112)])
        plsc.subcore_barrier()
        _row_pipeline(e2s, e2d, tbl, acc, idxg, idxd, rows, sem_g, sem_s,
                      base, 2, DEPTH1)
        plsc.subcore_barrier()
        for k in range(9):
            pltpu.sync_copy(acc.at[pl.ds(r0 + k * 128, 128)], rows.at[0])
            pltpu.sync_copy(rows.at[0], out.at[cid, p, pl.ds(r0 + k * 128, 128)])
        pltpu.sync_copy(acc.at[pl.ds(r0 + 1152, 112)],
                        rows.at[0, pl.ds(0, 112)])
        pltpu.sync_copy(rows.at[0, pl.ds(0, 112)],
                        out.at[cid, p, pl.ds(r0 + 1152, 112)])
        plsc.subcore_barrier()


def _sc_agg1(e2s, e2d, t0, t1, t2, t3, z64):
    return pl.kernel(
        _agg1_body,
        out_type=jax.ShapeDtypeStruct((2, 4, NE2, 64), f32),
        mesh=_mesh(),
        scratch_types=[
            pltpu.VMEM((GROUP, 128), i32),
            pltpu.VMEM((GROUP, 128), i32),
            pltpu.VMEM((DEPTH1, 128, 64), f32),
            pltpu.VMEM_SHARED((NE2, 64), f32),
            pltpu.SemaphoreType.DMA((DEPTH1,)),
            pltpu.SemaphoreType.DMA((DEPTH1,)),
        ],
        **_SC_PARAMS,
    )(e2s, e2d, t0, t1, t2, t3, z64)


def _agg2_body(srce, dste, q, z48, out, idxg, idxd, rows, st128, st120,
               acc, sem_g, sem_s):
    cid = lax.axis_index("c")
    sid = lax.axis_index("s")
    pltpu.sync_copy(z48, st128)
    pltpu.sync_copy(z48.at[pl.ds(0, 120)], st120)
    r0 = sid * STR2
    for k in range(4):
        pltpu.sync_copy(st128, acc.at[pl.ds(r0 + k * 128, 128)])
    pltpu.sync_copy(st120, acc.at[pl.ds(r0 + 512, 120)])
    plsc.subcore_barrier()
    base = cid * 1280 + sid * 80
    _row_pipeline(srce, dste, q, acc, idxg, idxd, rows, sem_g, sem_s, base, 2,
                  DEPTH2)
    plsc.subcore_barrier()
    for k in range(4):
        pltpu.sync_copy(acc.at[pl.ds(r0 + k * 128, 128)], st128)
        pltpu.sync_copy(st128, out.at[cid, pl.ds(r0 + k * 128, 128)])
    pltpu.sync_copy(acc.at[pl.ds(r0 + 512, 120)], st120)
    pltpu.sync_copy(st120, out.at[cid, pl.ds(r0 + 512, 120)])


def _sc_agg2(srce, dste, q, z48):
    return pl.kernel(
        _agg2_body,
        out_type=jax.ShapeDtypeStruct((2, NP, 48), f32),
        mesh=_mesh(),
        scratch_types=[
            pltpu.VMEM((GROUP, 128), i32),
            pltpu.VMEM((GROUP, 128), i32),
            pltpu.VMEM((DEPTH2, 128, 48), f32),
            pltpu.VMEM((128, 48), f32),
            pltpu.VMEM((120, 48), f32),
            pltpu.VMEM_SHARED((NP, 48), f32),
            pltpu.SemaphoreType.DMA((DEPTH2,)),
            pltpu.SemaphoreType.DMA((DEPTH2,)),
        ],
        **_SC_PARAMS,
    )(srce, dste, q, z48)


def kernel(x, edge_index, is_reversed, W_st1, b_st1, W_ts1, b_ts1, W_1, b_1,
           W_2, b_2):
    src = edge_index[0].astype(i32)
    dst = edge_index[1].astype(i32)
    rev = is_reversed.astype(i32)

    d3a, d3b, e2s, e2d = _tc_indices(
        src.reshape(2500, 128), dst.reshape(2500, 128), rev.reshape(2500, 128))
    arh = jnp.arange(LENH - 2 * E, dtype=i32)
    dst3 = jnp.concatenate(
        [d3a.ravel(), d3b.ravel(), 3 * N + arh % 64]).reshape(LENH // 128, 128)
    are = jnp.arange(LENE - E, dtype=i32)
    e2sp = jnp.concatenate([e2s.ravel(), are % 64]).reshape(LENE // 128, 128)
    e2dp = jnp.concatenate(
        [e2d.ravel(), 2 * N + are % 64]).reshape(LENE // 128, 128)
    srce = jnp.concatenate([src, are % 64]).reshape(LENE // 128, 128)
    dste = jnp.concatenate([dst, N + are % 64]).reshape(LENE // 128, 128)

    ones128 = jnp.ones((128,), f32)
    zh = jnp.zeros((STRH,), f32)
    z64 = jnp.zeros((128, 64), f32)
    z48 = jnp.zeros((128, 48), f32)

    wcat = jnp.concatenate([W_st1, W_1, W_ts1], axis=1)
    xp = jnp.pad(x, ((0, NP - N), (0, 0)))
    h = _tc_matmul(xp, wcat)
    h33 = h.reshape(NP, 3, 128)

    hist = _sc_hist(dst3, ones128, zh)
    cnt3 = jnp.pad(hist.reshape(2, NRH), ((0, 0), (0, N3 - NRH))).reshape(
        2, NP, 3, 1)
    t0, t1, t2, t3, dis3 = _tc_tables(cnt3, h33)
    tabs = [t.reshape(NE2, 64) for t in (t0, t1, t2, t3)]

    agg = _sc_agg1(e2sp, e2dp, *tabs, z64)
    parts1 = [agg[cid, g].reshape(NP, 2, 64) for cid in range(2)
              for g in range(4)]

    bcat = jnp.stack([b_st1, b_1, b_ts1])
    w2p = jnp.pad(
        jnp.concatenate([W_2[0:128], W_2[256:384], W_2[128:256]], axis=0),
        ((0, 0), (0, 8))).reshape(3, 128, 48)

    p, q = _tc_l2(parts1, h33, dis3, bcat, w2p)

    parts = _sc_agg2(srce, dste, q, z48)

    out = _tc_out(parts, p, dis3[:, 1, :], b_2.reshape(1, 40))
    return out[:N]
